# baseline (speedup 1.0000x reference)
import jax
import jax.numpy as jnp
from jax.experimental import pallas as pl
from jax.experimental.pallas import tpu as pltpu

E = 8
TOP_K = 2
D_MODEL = 768
D_FF = 3072
T = 2048
NA = T * TOP_K
TILE = 256
NT = (NA + E * (TILE - 1) + TILE - 1) // TILE


def _router_kernel(x_ref, wr_ref, probs_ref, pos_ref, counts_ref):
    x = x_ref[...]
    logits = jnp.dot(x, wr_ref[...], preferred_element_type=jnp.float32)
    m = jnp.max(logits, axis=-1, keepdims=True)
    ex = jnp.exp(logits - m)
    probs = ex / jnp.sum(ex, axis=-1, keepdims=True)
    probs_ref[...] = probs

    iota = jax.lax.broadcasted_iota(jnp.int32, probs.shape, 1)
    p1 = jnp.max(probs, axis=-1, keepdims=True)
    i1 = jnp.min(jnp.where(probs == p1, iota, E), axis=-1, keepdims=True)
    masked = jnp.where(iota == i1, -jnp.inf, probs)
    p2 = jnp.max(masked, axis=-1, keepdims=True)
    i2 = jnp.min(jnp.where(masked == p2, iota, E), axis=-1, keepdims=True)
    o1 = (iota == i1).astype(jnp.float32)
    o2 = (iota == i2).astype(jnp.float32)

    a = jnp.concatenate([o1, o2], axis=0)

    incl = a
    s = 1
    while s < NA:
        shifted = jnp.concatenate(
            [jnp.zeros((s, E), jnp.float32),
             jax.lax.slice(incl, (0, 0), (NA - s, E))], axis=0)
        incl = incl + shifted
        s *= 2
    excl = incl - a
    counts = jax.lax.slice(incl, (NA - 1, 0), (NA, E))
    counts_ref[...] = counts

    padded = jnp.floor((counts + (TILE - 1)) * (1.0 / TILE)) * TILE
    pincl = padded
    s = 1
    while s < E:
        shifted = jnp.concatenate(
            [jnp.zeros((1, s), jnp.float32),
             jax.lax.slice(pincl, (0, 0), (1, E - s))], axis=1)
        pincl = pincl + shifted
        s *= 2
    poffset = pincl - padded

    rank = jnp.sum(excl * a, axis=1, keepdims=True)
    offs = jnp.sum(poffset * a, axis=1, keepdims=True)
    pos_ref[...] = rank + offs


def _scatter_kernel(pos_ref, perm_ref):
    b = pl.program_id(0)
    pos = pos_ref[...].astype(jnp.int32)
    slot = jax.lax.broadcasted_iota(jnp.int32, (NA, TILE), 1) + b * TILE
    mask = (pos == slot).astype(jnp.float32)
    tokj = jax.lax.broadcasted_iota(jnp.int32, (NA, 1), 0)
    tok = jnp.where(tokj < T, tokj, tokj - T)
    lo = (tok % 256).astype(jnp.float32)
    hi = (tok // 256).astype(jnp.float32)
    vals = jnp.concatenate([lo, hi, jnp.ones_like(lo)], axis=1)
    res = jax.lax.dot_general(mask, vals, (((0,), (0,)), ((), ())),
                              preferred_element_type=jnp.float32)
    tokv = (jax.lax.slice(res, (0, 0), (TILE, 1))
            + 256.0 * jax.lax.slice(res, (0, 1), (TILE, 2)))
    hit = jax.lax.slice(res, (0, 2), (TILE, 3))
    perm_ref[...] = (tokv + (1.0 - hit) * T)[None]


def _ffn_kernel(meta_ref, x_ref, probs_ref, perm_ref, win_ref, wout_ref, out_ref):
    i = pl.program_id(0)

    @pl.when(i == 0)
    def _():
        out_ref[...] = jnp.zeros_like(out_ref)

    n_active = meta_ref[NT]

    @pl.when(i < n_active)
    def _():
        perm = perm_ref[0].astype(jnp.int32)
        tok_iota = jax.lax.broadcasted_iota(jnp.int32, (TILE, T), 1)
        m = (perm == tok_iota).astype(jnp.float32)
        xt = jnp.dot(m, x_ref[...], preferred_element_type=jnp.float32)
        h = jnp.dot(xt, win_ref[0], preferred_element_type=jnp.float32)
        h = jnp.maximum(h, 0.0)
        y = jnp.dot(h, wout_ref[0], preferred_element_type=jnp.float32)
        p = probs_ref[...]
        ph = p.astype(jnp.bfloat16).astype(jnp.float32)
        pg = (jnp.dot(m, ph, preferred_element_type=jnp.float32)
              + jnp.dot(m, p - ph, preferred_element_type=jnp.float32))
        e = meta_ref[i]
        col = jax.lax.broadcasted_iota(jnp.int32, (TILE, E), 1)
        g = jnp.sum(jnp.where(col == e, pg, 0.0), axis=1, keepdims=True)
        out_ref[...] += jax.lax.dot_general(
            m, y * g, (((0,), (0,)), ((), ())),
            preferred_element_type=jnp.float32)


@jax.jit
def kernel(input_batch, W_router, W_in, W_out):
    b, s, d = input_batch.shape
    x = input_batch.reshape(-1, d)

    probs, pos, counts = pl.pallas_call(
        _router_kernel,
        out_shape=[
            jax.ShapeDtypeStruct((T, E), jnp.float32),
            jax.ShapeDtypeStruct((NA, 1), jnp.float32),
            jax.ShapeDtypeStruct((1, E), jnp.float32),
        ],
    )(x, W_router)

    perm = pl.pallas_call(
        _scatter_kernel,
        grid=(NT,),
        in_specs=[pl.BlockSpec((NA, 1), lambda b: (0, 0))],
        out_specs=pl.BlockSpec((1, TILE, 1), lambda b: (b, 0, 0)),
        out_shape=jax.ShapeDtypeStruct((NT, TILE, 1), jnp.float32),
    )(pos)

    c = counts[0].astype(jnp.int32)
    padded = ((c + TILE - 1) // TILE) * TILE
    cum = jnp.cumsum(padded)
    n_active = cum[E - 1] // TILE
    starts = jnp.arange(NT, dtype=jnp.int32) * TILE
    e_tile = jnp.sum((cum[None, :] <= starts[:, None]).astype(jnp.int32), axis=1)
    last_e = jnp.take(e_tile, n_active - 1)
    e_tile = jnp.where(jnp.arange(NT) < n_active, e_tile, last_e)
    meta = jnp.concatenate([e_tile, n_active[None]]).astype(jnp.int32)

    out = pl.pallas_call(
        _ffn_kernel,
        grid_spec=pltpu.PrefetchScalarGridSpec(
            num_scalar_prefetch=1,
            grid=(NT,),
            in_specs=[
                pl.BlockSpec((T, D_MODEL), lambda i, m: (0, 0)),
                pl.BlockSpec((T, E), lambda i, m: (0, 0)),
                pl.BlockSpec((1, TILE, 1), lambda i, m: (i, 0, 0)),
                pl.BlockSpec((1, D_MODEL, D_FF), lambda i, m: (m[i], 0, 0)),
                pl.BlockSpec((1, D_FF, D_MODEL), lambda i, m: (m[i], 0, 0)),
            ],
            out_specs=pl.BlockSpec((T, D_MODEL), lambda i, m: (0, 0)),
        ),
        out_shape=jax.ShapeDtypeStruct((T, D_MODEL), jnp.float32),
    )(meta, x, probs, perm, W_in, W_out)

    return out.reshape(b, s, d)


# speedup vs baseline: 1.0427x; 1.0427x over previous
import jax
import jax.numpy as jnp
from jax.experimental import pallas as pl
from jax.experimental.pallas import tpu as pltpu

E = 8
TOP_K = 2
D_MODEL = 768
D_FF = 3072
T = 2048
NA = T * TOP_K
TILE = 256
NT = (NA + E * (TILE - 1) + TILE - 1) // TILE


def _router_kernel(x_ref, wr_ref, probs_ref, pos_ref, counts_ref):
    x = x_ref[...]
    logits = jnp.dot(x, wr_ref[...], preferred_element_type=jnp.float32)
    m = jnp.max(logits, axis=-1, keepdims=True)
    ex = jnp.exp(logits - m)
    probs = ex / jnp.sum(ex, axis=-1, keepdims=True)
    probs_ref[...] = probs

    iota = jax.lax.broadcasted_iota(jnp.int32, probs.shape, 1)
    p1 = jnp.max(probs, axis=-1, keepdims=True)
    i1 = jnp.min(jnp.where(probs == p1, iota, E), axis=-1, keepdims=True)
    masked = jnp.where(iota == i1, -jnp.inf, probs)
    p2 = jnp.max(masked, axis=-1, keepdims=True)
    i2 = jnp.min(jnp.where(masked == p2, iota, E), axis=-1, keepdims=True)
    o1 = (iota == i1).astype(jnp.float32)
    o2 = (iota == i2).astype(jnp.float32)

    a = jnp.concatenate([o1, o2], axis=0)

    incl = a
    s = 1
    while s < NA:
        shifted = jnp.concatenate(
            [jnp.zeros((s, E), jnp.float32),
             jax.lax.slice(incl, (0, 0), (NA - s, E))], axis=0)
        incl = incl + shifted
        s *= 2
    excl = incl - a
    counts = jax.lax.slice(incl, (NA - 1, 0), (NA, E))
    counts_ref[...] = counts

    padded = jnp.floor((counts + (TILE - 1)) * (1.0 / TILE)) * TILE
    pincl = padded
    s = 1
    while s < E:
        shifted = jnp.concatenate(
            [jnp.zeros((1, s), jnp.float32),
             jax.lax.slice(pincl, (0, 0), (1, E - s))], axis=1)
        pincl = pincl + shifted
        s *= 2
    poffset = pincl - padded

    rank = jnp.sum(excl * a, axis=1, keepdims=True)
    offs = jnp.sum(poffset * a, axis=1, keepdims=True)
    pos_ref[...] = rank + offs


def _scatter_kernel(pos_ref, perm_ref):
    b = pl.program_id(0)
    pos = pos_ref[...].astype(jnp.int32)
    slot = jax.lax.broadcasted_iota(jnp.int32, (NA, TILE), 1) + b * TILE
    mask = (pos == slot).astype(jnp.float32)
    tokj = jax.lax.broadcasted_iota(jnp.int32, (NA, 1), 0)
    tok = jnp.where(tokj < T, tokj, tokj - T)
    lo = (tok % 256).astype(jnp.float32)
    hi = (tok // 256).astype(jnp.float32)
    vals = jnp.concatenate([lo, hi, jnp.ones_like(lo)], axis=1)
    res = jax.lax.dot_general(mask, vals, (((0,), (0,)), ((), ())),
                              preferred_element_type=jnp.float32)
    tokv = (jax.lax.slice(res, (0, 0), (TILE, 1))
            + 256.0 * jax.lax.slice(res, (0, 1), (TILE, 2)))
    hit = jax.lax.slice(res, (0, 2), (TILE, 3))
    perm_ref[...] = (tokv + (1.0 - hit) * T)[None]


def _ffn_kernel(meta_ref, x_ref, probs_ref, perm_ref, win_ref, wout_ref, out_ref):
    i = pl.program_id(0)

    @pl.when(i == 0)
    def _():
        out_ref[...] = jnp.zeros_like(out_ref)

    n_active = meta_ref[NT]

    @pl.when(i < n_active)
    def _():
        perm = perm_ref[0].astype(jnp.int32)
        tok_iota = jax.lax.broadcasted_iota(jnp.int32, (TILE, T), 1)
        m = (perm == tok_iota).astype(jnp.float32)
        xt = jnp.dot(m, x_ref[...], preferred_element_type=jnp.float32)
        h = jnp.dot(xt, win_ref[0], preferred_element_type=jnp.float32)
        h = jnp.maximum(h, 0.0)
        y = jnp.dot(h, wout_ref[0], preferred_element_type=jnp.float32)
        pg = jnp.dot(m, probs_ref[...], preferred_element_type=jnp.float32)
        e = meta_ref[i]
        col = jax.lax.broadcasted_iota(jnp.int32, (TILE, E), 1)
        g = jnp.sum(jnp.where(col == e, pg, 0.0), axis=1, keepdims=True)
        out_ref[...] += jax.lax.dot_general(
            m, y * g, (((0,), (0,)), ((), ())),
            preferred_element_type=jnp.float32)


@jax.jit
def kernel(input_batch, W_router, W_in, W_out):
    b, s, d = input_batch.shape
    x = input_batch.reshape(-1, d)

    probs, pos, counts = pl.pallas_call(
        _router_kernel,
        out_shape=[
            jax.ShapeDtypeStruct((T, E), jnp.float32),
            jax.ShapeDtypeStruct((NA, 1), jnp.float32),
            jax.ShapeDtypeStruct((1, E), jnp.float32),
        ],
    )(x, W_router)

    perm = pl.pallas_call(
        _scatter_kernel,
        grid=(NT,),
        in_specs=[pl.BlockSpec((NA, 1), lambda b: (0, 0))],
        out_specs=pl.BlockSpec((1, TILE, 1), lambda b: (b, 0, 0)),
        out_shape=jax.ShapeDtypeStruct((NT, TILE, 1), jnp.float32),
    )(pos)

    c = counts[0].astype(jnp.int32)
    padded = ((c + TILE - 1) // TILE) * TILE
    cum = jnp.cumsum(padded)
    n_active = cum[E - 1] // TILE
    starts = jnp.arange(NT, dtype=jnp.int32) * TILE
    e_tile = jnp.sum((cum[None, :] <= starts[:, None]).astype(jnp.int32), axis=1)
    last_e = jnp.take(e_tile, n_active - 1)
    e_tile = jnp.where(jnp.arange(NT) < n_active, e_tile, last_e)
    meta = jnp.concatenate([e_tile, n_active[None]]).astype(jnp.int32)

    out = pl.pallas_call(
        _ffn_kernel,
        grid_spec=pltpu.PrefetchScalarGridSpec(
            num_scalar_prefetch=1,
            grid=(NT,),
            in_specs=[
                pl.BlockSpec((T, D_MODEL), lambda i, m: (0, 0)),
                pl.BlockSpec((T, E), lambda i, m: (0, 0)),
                pl.BlockSpec((1, TILE, 1), lambda i, m: (i, 0, 0)),
                pl.BlockSpec((1, D_MODEL, D_FF), lambda i, m: (m[i], 0, 0)),
                pl.BlockSpec((1, D_FF, D_MODEL), lambda i, m: (m[i], 0, 0)),
            ],
            out_specs=pl.BlockSpec((T, D_MODEL), lambda i, m: (0, 0)),
        ),
        out_shape=jax.ShapeDtypeStruct((T, D_MODEL), jnp.float32),
    )(meta, x, probs, perm, W_in, W_out)

    return out.reshape(b, s, d)


# speedup vs baseline: 1.0594x; 1.0161x over previous
import functools

import jax
import jax.numpy as jnp
from jax.experimental import pallas as pl
from jax.experimental.pallas import tpu as pltpu
from jax.experimental.pallas import tpu_sc as plsc

E = 8
TOP_K = 2
D_MODEL = 768
D_FF = 3072
T = 2048
NA = T * TOP_K
TILE = 256
NT = (NA + E * (TILE - 1) + TILE - 1) // TILE
P = NT * TILE

SC_SUBCORES = 16
SC_LANES = 16


def _router_kernel(x_ref, wr_ref, probs_ref, pos_ref, counts_ref):
    x = x_ref[...]
    logits = jnp.dot(x, wr_ref[...], preferred_element_type=jnp.float32)
    m = jnp.max(logits, axis=-1, keepdims=True)
    ex = jnp.exp(logits - m)
    probs = ex / jnp.sum(ex, axis=-1, keepdims=True)
    probs_ref[...] = probs

    iota = jax.lax.broadcasted_iota(jnp.int32, probs.shape, 1)
    p1 = jnp.max(probs, axis=-1, keepdims=True)
    i1 = jnp.min(jnp.where(probs == p1, iota, E), axis=-1, keepdims=True)
    masked = jnp.where(iota == i1, -jnp.inf, probs)
    p2 = jnp.max(masked, axis=-1, keepdims=True)
    i2 = jnp.min(jnp.where(masked == p2, iota, E), axis=-1, keepdims=True)
    o1 = (iota == i1).astype(jnp.float32)
    o2 = (iota == i2).astype(jnp.float32)

    counts = jnp.sum(o1, axis=0, keepdims=True) + jnp.sum(o2, axis=0, keepdims=True)
    counts_ref[...] = counts

    padded = jnp.floor((counts + (TILE - 1)) * (1.0 / TILE)) * TILE
    pincl = padded
    s = 1
    while s < E:
        shifted = jnp.concatenate(
            [jnp.zeros((1, s), jnp.float32),
             jax.lax.slice(pincl, (0, 0), (1, E - s))], axis=1)
        pincl = pincl + shifted
        s *= 2
    poffset = pincl - padded

    rb = jax.lax.broadcasted_iota(jnp.int32, (TILE, TILE), 0)
    cb = jax.lax.broadcasted_iota(jnp.int32, (TILE, TILE), 1)
    tri = (rb > cb).astype(jnp.float32)
    carry = jnp.zeros((1, E), dtype=jnp.float32)
    nblk = T // TILE
    for half, o in enumerate((o1, o2)):
        for b in range(nblk):
            sl = jax.lax.slice(o, (b * TILE, 0), ((b + 1) * TILE, E))
            excl = jnp.dot(tri, sl, preferred_element_type=jnp.float32) + carry
            rank = jnp.sum(excl * sl, axis=1, keepdims=True)
            offs = jnp.sum(poffset * sl, axis=1, keepdims=True)
            pos_ref[pl.ds(half * T + b * TILE, TILE), :] = rank + offs
            carry = carry + jnp.sum(sl, axis=0, keepdims=True)


def _perm_sc_kernel(pos_hbm, init_hbm, out_hbm, idx_v, val_v):
    sid = jax.lax.axis_index("s")
    ini = P // SC_SUBCORES
    pltpu.sync_copy(init_hbm.at[pl.ds(sid * ini, ini)],
                    out_hbm.at[pl.ds(sid * ini, ini)])
    plsc.subcore_barrier()
    per_w = NA // SC_SUBCORES
    base = sid * per_w
    for j in range(per_w // SC_LANES):
        pltpu.sync_copy(pos_hbm.at[pl.ds(base + j * SC_LANES, SC_LANES)], idx_v)
        a = base + j * SC_LANES + jax.lax.iota(jnp.int32, SC_LANES)
        tok = jnp.where(a >= T, a - T, a)
        val_v[...] = tok
        pltpu.sync_copy(val_v, out_hbm.at[idx_v])


_perm_sc = functools.partial(
    pl.kernel,
    out_type=jax.ShapeDtypeStruct((P,), jnp.int32),
    mesh=plsc.VectorSubcoreMesh(
        core_axis_name="c", subcore_axis_name="s", num_cores=1),
    scratch_types=[
        pltpu.VMEM((SC_LANES,), jnp.int32),
        pltpu.VMEM((SC_LANES,), jnp.int32),
    ],
)(_perm_sc_kernel)


def _ffn_kernel(meta_ref, x_ref, probs_ref, perm_ref, win_ref, wout_ref, out_ref):
    i = pl.program_id(0)

    @pl.when(i == 0)
    def _():
        out_ref[...] = jnp.zeros_like(out_ref)

    n_active = meta_ref[NT]

    @pl.when(i < n_active)
    def _():
        perm = perm_ref[0]
        tok_iota = jax.lax.broadcasted_iota(jnp.int32, (T, TILE), 0)
        mt = (perm == tok_iota).astype(jnp.float32)
        xt = jax.lax.dot_general(mt, x_ref[...], (((0,), (0,)), ((), ())),
                                 preferred_element_type=jnp.float32)
        h = jnp.dot(xt, win_ref[0], preferred_element_type=jnp.float32)
        h = jnp.maximum(h, 0.0)
        y = jnp.dot(h, wout_ref[0], preferred_element_type=jnp.float32)
        pg = jax.lax.dot_general(mt, probs_ref[...], (((0,), (0,)), ((), ())),
                                 preferred_element_type=jnp.float32)
        e = meta_ref[i]
        col = jax.lax.broadcasted_iota(jnp.int32, (TILE, E), 1)
        g = jnp.sum(jnp.where(col == e, pg, 0.0), axis=1, keepdims=True)
        out_ref[...] += jnp.dot(mt, y * g, preferred_element_type=jnp.float32)


@jax.jit
def kernel(input_batch, W_router, W_in, W_out):
    b, s, d = input_batch.shape
    x = input_batch.reshape(-1, d)

    probs, pos, counts = pl.pallas_call(
        _router_kernel,
        out_shape=[
            jax.ShapeDtypeStruct((T, E), jnp.float32),
            jax.ShapeDtypeStruct((NA, 1), jnp.float32),
            jax.ShapeDtypeStruct((1, E), jnp.float32),
        ],
    )(x, W_router)

    sentinel = jnp.full((P,), T, dtype=jnp.int32)
    perm = _perm_sc(pos[:, 0].astype(jnp.int32), sentinel)
    perm = perm.reshape(NT, 1, TILE)

    c = counts[0].astype(jnp.int32)
    padded = ((c + TILE - 1) // TILE) * TILE
    cum = jnp.cumsum(padded)
    n_active = cum[E - 1] // TILE
    starts = jnp.arange(NT, dtype=jnp.int32) * TILE
    e_tile = jnp.sum((cum[None, :] <= starts[:, None]).astype(jnp.int32), axis=1)
    last_e = jnp.take(e_tile, n_active - 1)
    e_tile = jnp.where(jnp.arange(NT) < n_active, e_tile, last_e)
    meta = jnp.concatenate([e_tile, n_active[None]]).astype(jnp.int32)

    out = pl.pallas_call(
        _ffn_kernel,
        grid_spec=pltpu.PrefetchScalarGridSpec(
            num_scalar_prefetch=1,
            grid=(NT,),
            in_specs=[
                pl.BlockSpec((T, D_MODEL), lambda i, m: (0, 0)),
                pl.BlockSpec((T, E), lambda i, m: (0, 0)),
                pl.BlockSpec((1, 1, TILE), lambda i, m: (i, 0, 0)),
                pl.BlockSpec((1, D_MODEL, D_FF), lambda i, m: (m[i], 0, 0)),
                pl.BlockSpec((1, D_FF, D_MODEL), lambda i, m: (m[i], 0, 0)),
            ],
            out_specs=pl.BlockSpec((T, D_MODEL), lambda i, m: (0, 0)),
        ),
        out_shape=jax.ShapeDtypeStruct((T, D_MODEL), jnp.float32),
    )(meta, x, probs, perm, W_in, W_out)

    return out.reshape(b, s, d)
